# baseline (speedup 1.0000x reference)
import functools

import jax
import jax.numpy as jnp
from jax import lax
from jax.experimental import pallas as pl
from jax.experimental.pallas import tpu as pltpu
from jax.experimental.pallas import tpu_sc as plsc

N = 1000000
D = 64
B = 16384
ENTITY_COFF = 0.1

NC = 2
NS = 16
L = 16
DPC = D // NC
EPT = B // NS
GROUPS = EPT // L


def _sc_body(tableT, pairsT, part_out,
             praw, pidx, vals5, acf, arg, row_sh, fsem, gsem):
    c = lax.axis_index("c")
    s = lax.axis_index("s")
    ebase = s * EPT
    pltpu.sync_copy(pairsT.at[:, pl.ds(ebase, EPT)], praw)
    for k in range(5):
        for g in range(GROUPS):
            pidx[pl.ds(k * EPT + g * L, L)] = praw[k, pl.ds(g * L, L)]
    zeros = jnp.zeros((L,), jnp.float32)
    for g in range(GROUPS):
        acf[pl.ds(g * L, L)] = zeros
        arg[pl.ds(g * L, L)] = zeros

    FW = 62464
    FT = N - 15 * FW

    def seg_dma(d, off, width):
        cols = pl.ds(off, width)
        return pltpu.make_async_copy(
            tableT.at[pl.ds(d, 1), cols], row_sh.at[:, cols], fsem)

    def dstep(i, carry):
        d = c * DPC + i
        @pl.when(s < 15)
        def _():
            seg_dma(d, s * FW, FW).start()
            seg_dma(d, s * FW, FW).wait()

        @pl.when(s == 15)
        def _():
            seg_dma(d, 15 * FW, FT).start()
            seg_dma(d, 15 * FW, FT).wait()
        plsc.subcore_barrier()
        pltpu.async_copy(row_sh.at[0].at[pidx], vals5, gsem).wait()

        def gstep(g, carry2):
            u = vals5[pl.ds(g * L, L)]
            p = vals5[pl.ds(EPT + g * L, L)]
            n = vals5[pl.ds(2 * EPT + g * L, L)]
            ep = vals5[pl.ds(3 * EPT + g * L, L)]
            en = vals5[pl.ds(4 * EPT + g * L, L)]
            off = pl.ds(g * L, L)
            acf[off] += u * (p - n)
            arg[off] += (en - ep) * (p + p - ep - en)
            return carry2

        lax.fori_loop(0, GROUPS, gstep, 0, unroll=4)
        plsc.subcore_barrier()
        return carry

    lax.fori_loop(0, DPC, dstep, 0)
    obase = c * (2 * B) + ebase
    pltpu.sync_copy(acf, part_out.at[pl.ds(obase, EPT)])
    pltpu.sync_copy(arg, part_out.at[pl.ds(obase + B, EPT)])


_sc_dloop = functools.partial(
    pl.kernel,
    mesh=plsc.VectorSubcoreMesh(core_axis_name="c", subcore_axis_name="s"),
    out_type=jax.ShapeDtypeStruct((4 * B,), jnp.float32),
    scratch_types=[
        pltpu.VMEM((5, EPT), jnp.int32),
        pltpu.VMEM((5 * EPT,), jnp.int32),
        pltpu.VMEM((5 * EPT,), jnp.float32),
        pltpu.VMEM((EPT,), jnp.float32),
        pltpu.VMEM((EPT,), jnp.float32),
        pltpu.VMEM_SHARED((1, N), jnp.float32),
        pltpu.SemaphoreType.DMA,
        pltpu.SemaphoreType.DMA,
    ],
    compiler_params=pltpu.CompilerParams(needs_layout_passes=False),
)(_sc_body)


def _loss_body(part_ref, out_ref):
    xcf = part_ref[0, :, :] + part_ref[2, :, :]
    xreg = part_ref[1, :, :] + part_ref[3, :, :]

    def neg_logsig_sum(x):
        m = jnp.minimum(x, 0.0)
        z = jnp.exp(-jnp.abs(x))
        return jnp.sum(jnp.log1p(z) - m)

    out_ref[0, 0] = (neg_logsig_sum(xcf)
                     + ENTITY_COFF * neg_logsig_sum(xreg))


_tc_loss = pl.pallas_call(
    _loss_body,
    out_shape=jax.ShapeDtypeStruct((1, 1), jnp.float32),
    out_specs=pl.BlockSpec(memory_space=pltpu.SMEM),
)


@jax.jit
def kernel(cached_repr, pos_neg_pair_t):
    tableT = cached_repr.T
    pairsT = pos_neg_pair_t.T
    part = _sc_dloop(tableT, pairsT)
    loss = _tc_loss(part.reshape(4, 128, 128))
    return loss[0, 0]


# speedup vs baseline: 1.1076x; 1.1076x over previous
import functools

import jax
import jax.numpy as jnp
from jax import lax
from jax.experimental import pallas as pl
from jax.experimental.pallas import tpu as pltpu
from jax.experimental.pallas import tpu_sc as plsc

N = 1000000
D = 64
B = 16384
ENTITY_COFF = 0.1

NC = 2
NS = 16
L = 16
DPC = D // NC
EPT = B // NS
GROUPS = EPT // L


PA = 687360
RB = 2 * PA
PB = N - PA
AFW = 43008
BFW = 19584


def _sc_body(tableT, pairsT, part_out,
             praw, pidx_e, pidx_o, vals5, acf, arg, arena, sa0, sa1, sb, gsem):
    c = lax.axis_index("c")
    s = lax.axis_index("s")
    ebase = s * EPT
    pltpu.sync_copy(pairsT.at[:, pl.ds(ebase, EPT)], praw)
    for k in range(5):
        for g in range(GROUPS):
            col = praw[k, pl.ds(g * L, L)]
            tail = col + (RB - PA)
            off = pl.ds(k * EPT + g * L, L)
            pidx_e[off] = jnp.where(col < PA, col, tail)
            pidx_o[off] = jnp.where(col < PA, col + PA, tail)
    zeros = jnp.zeros((L,), jnp.float32)
    for g in range(GROUPS):
        acf[pl.ds(g * L, L)] = zeros
        arg[pl.ds(g * L, L)] = zeros

    def fill_a(d, p):
        sa = sa0 if p == 0 else sa1

        def seg(off, width):
            return pltpu.make_async_copy(
                tableT.at[pl.ds(d, 1), pl.ds(off, width)],
                arena.at[:, pl.ds(p * PA + off, width)], sa)

        @pl.when(s < 15)
        def _():
            seg(s * AFW, AFW).start()

        @pl.when(s == 15)
        def _():
            seg(15 * AFW, PA - 15 * AFW).start()

    def wait_a(p):
        sa = sa0 if p == 0 else sa1

        @pl.when(s < 15)
        def _():
            pltpu.make_async_copy(
                tableT.at[pl.ds(0, 1), pl.ds(0, AFW)],
                arena.at[:, pl.ds(0, AFW)], sa).wait()

        @pl.when(s == 15)
        def _():
            pltpu.make_async_copy(
                tableT.at[pl.ds(0, 1), pl.ds(0, PA - 15 * AFW)],
                arena.at[:, pl.ds(0, PA - 15 * AFW)], sa).wait()

    def fill_b(d):
        def seg(off, width):
            return pltpu.make_async_copy(
                tableT.at[pl.ds(d, 1), pl.ds(PA + off, width)],
                arena.at[:, pl.ds(RB + off, width)], sb)

        @pl.when(s < 15)
        def _():
            seg(s * BFW, BFW).start()

        @pl.when(s == 15)
        def _():
            seg(15 * BFW, PB - 15 * BFW).start()

    def wait_b():
        @pl.when(s < 15)
        def _():
            pltpu.make_async_copy(
                tableT.at[pl.ds(0, 1), pl.ds(0, BFW)],
                arena.at[:, pl.ds(0, BFW)], sb).wait()

        @pl.when(s == 15)
        def _():
            w = PB - 15 * BFW
            pltpu.make_async_copy(
                tableT.at[pl.ds(0, 1), pl.ds(N - w, w)],
                arena.at[:, pl.ds(2 * PA + PB - w, w)], sb).wait()

    def phase(i, p, pidx):
        d = c * DPC + i
        fill_b(d)

        @pl.when(i + 1 < DPC)
        def _():
            fill_a(d + 1, 1 - p)
        wait_a(p)
        wait_b()
        plsc.subcore_barrier()
        pltpu.async_copy(arena.at[0].at[pidx], vals5, gsem).wait()

        def gstep(g, carry2):
            u = vals5[pl.ds(g * L, L)]
            pp = vals5[pl.ds(EPT + g * L, L)]
            n = vals5[pl.ds(2 * EPT + g * L, L)]
            ep = vals5[pl.ds(3 * EPT + g * L, L)]
            en = vals5[pl.ds(4 * EPT + g * L, L)]
            off = pl.ds(g * L, L)
            acf[off] += u * (pp - n)
            arg[off] += (en - ep) * (pp + pp - ep - en)
            return carry2

        lax.fori_loop(0, GROUPS, gstep, 0, unroll=4)
        plsc.subcore_barrier()

    fill_a(c * DPC, 0)

    def dstep2(j, carry):
        phase(2 * j, 0, pidx_e)
        phase(2 * j + 1, 1, pidx_o)
        return carry

    lax.fori_loop(0, DPC // 2, dstep2, 0)
    obase = c * (2 * B) + ebase
    pltpu.sync_copy(acf, part_out.at[pl.ds(obase, EPT)])
    pltpu.sync_copy(arg, part_out.at[pl.ds(obase + B, EPT)])


_sc_dloop = functools.partial(
    pl.kernel,
    mesh=plsc.VectorSubcoreMesh(core_axis_name="c", subcore_axis_name="s"),
    out_type=jax.ShapeDtypeStruct((4 * B,), jnp.float32),
    scratch_types=[
        pltpu.VMEM((5, EPT), jnp.int32),
        pltpu.VMEM((5 * EPT,), jnp.int32),
        pltpu.VMEM((5 * EPT,), jnp.int32),
        pltpu.VMEM((5 * EPT,), jnp.float32),
        pltpu.VMEM((EPT,), jnp.float32),
        pltpu.VMEM((EPT,), jnp.float32),
        pltpu.VMEM_SHARED((1, 2 * PA + PB), jnp.float32),
        pltpu.SemaphoreType.DMA,
        pltpu.SemaphoreType.DMA,
        pltpu.SemaphoreType.DMA,
        pltpu.SemaphoreType.DMA,
    ],
    compiler_params=pltpu.CompilerParams(needs_layout_passes=False),
)(_sc_body)


def _loss_body(part_ref, out_ref):
    xcf = part_ref[0, :, :] + part_ref[2, :, :]
    xreg = part_ref[1, :, :] + part_ref[3, :, :]

    def neg_logsig_sum(x):
        m = jnp.minimum(x, 0.0)
        z = jnp.exp(-jnp.abs(x))
        return jnp.sum(jnp.log1p(z) - m)

    out_ref[0, 0] = (neg_logsig_sum(xcf)
                     + ENTITY_COFF * neg_logsig_sum(xreg))


_tc_loss = pl.pallas_call(
    _loss_body,
    out_shape=jax.ShapeDtypeStruct((1, 1), jnp.float32),
    out_specs=pl.BlockSpec(memory_space=pltpu.SMEM),
)


@jax.jit
def kernel(cached_repr, pos_neg_pair_t):
    tableT = cached_repr.T
    pairsT = pos_neg_pair_t.T
    part = _sc_dloop(tableT, pairsT)
    loss = _tc_loss(part.reshape(4, 128, 128))
    return loss[0, 0]


# speedup vs baseline: 1.1360x; 1.0256x over previous
import functools

import jax
import jax.numpy as jnp
from jax import lax
from jax.experimental import pallas as pl
from jax.experimental.pallas import tpu as pltpu
from jax.experimental.pallas import tpu_sc as plsc

N = 1000000
D = 64
B = 16384
ENTITY_COFF = 0.1

NC = 2
NS = 16
L = 16
DPC = D // NC
EPT = B // NS
GROUPS = EPT // L


PA = 687360
RB = 2 * PA
PB = N - PA
AFW = 43008
BFW = 19584


def _sc_body(tableT, pairsT, part_out,
             praw, pidx_e, pidx_o, vals5, acf, arg, arena, sa0, sa1, sb, gsem):
    c = lax.axis_index("c")
    s = lax.axis_index("s")
    ebase = s * EPT
    pltpu.sync_copy(pairsT.at[:, pl.ds(ebase, EPT)], praw)
    for k in range(5):
        for g in range(GROUPS):
            col = praw[k, pl.ds(g * L, L)]
            tail = col + (RB - PA)
            off = pl.ds(k * EPT + g * L, L)
            pidx_e[off] = jnp.where(col < PA, col, tail)
            pidx_o[off] = jnp.where(col < PA, col + PA, tail)
    zeros = jnp.zeros((L,), jnp.float32)
    for g in range(GROUPS):
        acf[pl.ds(g * L, L)] = zeros
        arg[pl.ds(g * L, L)] = zeros

    def fill_a(d, p):
        sa = sa0 if p == 0 else sa1

        def seg(off, width):
            return pltpu.make_async_copy(
                tableT.at[pl.ds(d, 1), pl.ds(off, width)],
                arena.at[:, pl.ds(p * PA + off, width)], sa)

        @pl.when(s < 15)
        def _():
            seg(s * AFW, AFW).start()

        @pl.when(s == 15)
        def _():
            seg(15 * AFW, PA - 15 * AFW).start()

    def wait_a(p):
        sa = sa0 if p == 0 else sa1

        @pl.when(s < 15)
        def _():
            pltpu.make_async_copy(
                tableT.at[pl.ds(0, 1), pl.ds(0, AFW)],
                arena.at[:, pl.ds(0, AFW)], sa).wait()

        @pl.when(s == 15)
        def _():
            pltpu.make_async_copy(
                tableT.at[pl.ds(0, 1), pl.ds(0, PA - 15 * AFW)],
                arena.at[:, pl.ds(0, PA - 15 * AFW)], sa).wait()

    def fill_b(d):
        def seg(off, width):
            return pltpu.make_async_copy(
                tableT.at[pl.ds(d, 1), pl.ds(PA + off, width)],
                arena.at[:, pl.ds(RB + off, width)], sb)

        @pl.when(s < 15)
        def _():
            seg(s * BFW, BFW).start()

        @pl.when(s == 15)
        def _():
            seg(15 * BFW, PB - 15 * BFW).start()

    def wait_b():
        @pl.when(s < 15)
        def _():
            pltpu.make_async_copy(
                tableT.at[pl.ds(0, 1), pl.ds(0, BFW)],
                arena.at[:, pl.ds(0, BFW)], sb).wait()

        @pl.when(s == 15)
        def _():
            w = PB - 15 * BFW
            pltpu.make_async_copy(
                tableT.at[pl.ds(0, 1), pl.ds(N - w, w)],
                arena.at[:, pl.ds(2 * PA + PB - w, w)], sb).wait()

    def phase(i, p, pidx):
        d = c * DPC + i
        wait_a(p)
        wait_b()
        plsc.subcore_barrier()
        pltpu.async_copy(arena.at[0].at[pidx], vals5, gsem).wait()
        plsc.subcore_barrier()

        @pl.when(i + 1 < DPC)
        def _():
            fill_b(d + 1)

        @pl.when(i + 2 < DPC)
        def _():
            fill_a(d + 2, p)

        def gstep(g, carry2):
            u = vals5[pl.ds(g * L, L)]
            pp = vals5[pl.ds(EPT + g * L, L)]
            n = vals5[pl.ds(2 * EPT + g * L, L)]
            ep = vals5[pl.ds(3 * EPT + g * L, L)]
            en = vals5[pl.ds(4 * EPT + g * L, L)]
            off = pl.ds(g * L, L)
            acf[off] += u * (pp - n)
            arg[off] += (en - ep) * (pp + pp - ep - en)
            return carry2

        lax.fori_loop(0, GROUPS, gstep, 0, unroll=4)

    fill_a(c * DPC, 0)
    fill_b(c * DPC)
    fill_a(c * DPC + 1, 1)

    def dstep2(j, carry):
        phase(2 * j, 0, pidx_e)
        phase(2 * j + 1, 1, pidx_o)
        return carry

    lax.fori_loop(0, DPC // 2, dstep2, 0)
    obase = c * (2 * B) + ebase
    pltpu.sync_copy(acf, part_out.at[pl.ds(obase, EPT)])
    pltpu.sync_copy(arg, part_out.at[pl.ds(obase + B, EPT)])


_sc_dloop = functools.partial(
    pl.kernel,
    mesh=plsc.VectorSubcoreMesh(core_axis_name="c", subcore_axis_name="s"),
    out_type=jax.ShapeDtypeStruct((4 * B,), jnp.float32),
    scratch_types=[
        pltpu.VMEM((5, EPT), jnp.int32),
        pltpu.VMEM((5 * EPT,), jnp.int32),
        pltpu.VMEM((5 * EPT,), jnp.int32),
        pltpu.VMEM((5 * EPT,), jnp.float32),
        pltpu.VMEM((EPT,), jnp.float32),
        pltpu.VMEM((EPT,), jnp.float32),
        pltpu.VMEM_SHARED((1, 2 * PA + PB), jnp.float32),
        pltpu.SemaphoreType.DMA,
        pltpu.SemaphoreType.DMA,
        pltpu.SemaphoreType.DMA,
        pltpu.SemaphoreType.DMA,
    ],
    compiler_params=pltpu.CompilerParams(needs_layout_passes=False),
)(_sc_body)


def _loss_body(part_ref, out_ref):
    xcf = part_ref[0, :, :] + part_ref[2, :, :]
    xreg = part_ref[1, :, :] + part_ref[3, :, :]

    def neg_logsig_sum(x):
        m = jnp.minimum(x, 0.0)
        z = jnp.exp(-jnp.abs(x))
        return jnp.sum(jnp.log1p(z) - m)

    out_ref[0, 0] = (neg_logsig_sum(xcf)
                     + ENTITY_COFF * neg_logsig_sum(xreg))


_tc_loss = pl.pallas_call(
    _loss_body,
    out_shape=jax.ShapeDtypeStruct((1, 1), jnp.float32),
    out_specs=pl.BlockSpec(memory_space=pltpu.SMEM),
)


@jax.jit
def kernel(cached_repr, pos_neg_pair_t):
    tableT = cached_repr.T
    pairsT = pos_neg_pair_t.T
    part = _sc_dloop(tableT, pairsT)
    loss = _tc_loss(part.reshape(4, 128, 128))
    return loss[0, 0]
